# baseline (speedup 1.0000x reference)
import functools

import jax
import jax.numpy as jnp
from jax import lax
from jax.experimental import pallas as pl
from jax.experimental.pallas import tpu as pltpu
from jax.experimental.pallas import tpu_sc as plsc

N_NODES = 10000
N_PAD = 10240
D = 128
G = 128
E = 320000
NC = 2
NS = 16
NW = NC * NS
CHUNK = 128
NCHUNK = 80
E_PAD = NW * NCHUNK * CHUNK
PAD_NODE = N_NODES
ROWS_PER_TILE = N_PAD // NS
BLK = 512
NBLK = N_PAD // BLK

_mesh = plsc.VectorSubcoreMesh(core_axis_name="c", subcore_axis_name="s")


@functools.partial(
    pl.kernel,
    mesh=_mesh,
    out_type=jax.ShapeDtypeStruct((NC, N_PAD), jnp.float32),
    scratch_types=[
        pltpu.VMEM((2, CHUNK), jnp.int32),
        pltpu.VMEM((2, CHUNK), jnp.int32),
        pltpu.VMEM((128,), jnp.float32),
        pltpu.VMEM((ROWS_PER_TILE,), jnp.float32),
        pltpu.VMEM_SHARED((N_PAD,), jnp.float32),
        pltpu.SemaphoreType.DMA,
        pltpu.SemaphoreType.DMA,
        pltpu.SemaphoreType.DMA,
        pltpu.SemaphoreType.DMA,
    ],
)
def _sc_degree(eidx_hbm, out_hbm, idx_a, idx_b, ones_v, zero_v, acc,
               si_a, si_b, ss_a, ss_b):
    c = lax.axis_index("c")
    s = lax.axis_index("s")
    wid = c * NS + s
    pltpu.async_copy(eidx_hbm.at[wid, 0], idx_a, si_a)
    pltpu.async_copy(eidx_hbm.at[wid, 1], idx_b, si_b)
    for k in range(8):
        ones_v[pl.ds(k * 16, 16)] = jnp.ones((16,), jnp.float32)

    def zbody(i, carry):
        zero_v[pl.ds(i * 16, 16)] = jnp.zeros((16,), jnp.float32)
        return carry

    lax.fori_loop(0, ROWS_PER_TILE // 16, zbody, 0)
    r0 = s * ROWS_PER_TILE
    pltpu.sync_copy(zero_v, acc.at[pl.ds(r0, ROWS_PER_TILE)])
    plsc.subcore_barrier()

    def body(i, carry):
        j = 2 * i
        pltpu.make_async_copy(eidx_hbm.at[wid, 0], idx_a, si_a).wait()
        pltpu.async_copy(ones_v.at[pl.ds(0, CHUNK)], acc.at[idx_a.at[1]],
                         ss_a, add=True)
        pltpu.make_async_copy(eidx_hbm.at[wid, 0], idx_b, si_b).wait()
        pltpu.async_copy(ones_v.at[pl.ds(0, CHUNK)], acc.at[idx_b.at[1]],
                         ss_b, add=True)
        pltpu.make_async_copy(ones_v.at[pl.ds(0, CHUNK)], acc.at[idx_a.at[1]],
                              ss_a).wait()

        @pl.when(j + 2 < NCHUNK)
        def _():
            pltpu.async_copy(eidx_hbm.at[wid, j + 2], idx_a, si_a)

        pltpu.make_async_copy(ones_v.at[pl.ds(0, CHUNK)], acc.at[idx_b.at[1]],
                              ss_b).wait()

        @pl.when(j + 3 < NCHUNK)
        def _():
            pltpu.async_copy(eidx_hbm.at[wid, j + 3], idx_b, si_b)

        return carry

    lax.fori_loop(0, NCHUNK // 2, body, 0)
    plsc.subcore_barrier()
    pltpu.sync_copy(acc.at[pl.ds(r0, ROWS_PER_TILE)],
                    out_hbm.at[c, pl.ds(r0, ROWS_PER_TILE)])


@functools.partial(
    pl.kernel,
    mesh=_mesh,
    out_type=jax.ShapeDtypeStruct((NC, N_PAD, D), jnp.float32),
    scratch_types=[
        pltpu.VMEM((2, CHUNK), jnp.int32),
        pltpu.VMEM((2, CHUNK), jnp.int32),
        pltpu.VMEM((2, CHUNK), jnp.int32),
        pltpu.VMEM((2, CHUNK), jnp.int32),
        pltpu.VMEM((CHUNK, D), jnp.float32),
        pltpu.VMEM((CHUNK, D), jnp.float32),
        pltpu.VMEM_SHARED((N_PAD, D), jnp.float32),
        pltpu.SemaphoreType.DMA,
        pltpu.SemaphoreType.DMA,
        pltpu.SemaphoreType.DMA,
        pltpu.SemaphoreType.DMA,
        pltpu.SemaphoreType.DMA,
        pltpu.SemaphoreType.DMA,
        pltpu.SemaphoreType.DMA,
        pltpu.SemaphoreType.DMA,
    ],
)
def _sc_aggregate(g_hbm, eidx_hbm, out_hbm,
                  i0, i1, i2, i3, r0b, r1b, acc,
                  sg0, sg1, ss0, ss1, si0, si1, si2, si3):
    c = lax.axis_index("c")
    s = lax.axis_index("s")
    wid = c * NS + s
    pltpu.async_copy(eidx_hbm.at[wid, 0], i0, si0)
    pltpu.async_copy(eidx_hbm.at[wid, 1], i1, si1)
    pltpu.async_copy(eidx_hbm.at[wid, 2], i2, si2)
    pltpu.async_copy(eidx_hbm.at[wid, 3], i3, si3)

    def zbody(i, carry):
        for k in range(8):
            r0b[i, pl.ds(k * 16, 16)] = jnp.zeros((16,), jnp.float32)
        return carry

    lax.fori_loop(0, CHUNK, zbody, 0)
    rbase = s * ROWS_PER_TILE
    for t in range(ROWS_PER_TILE // CHUNK):
        pltpu.sync_copy(r0b, acc.at[pl.ds(rbase + t * CHUNK, CHUNK)])
    plsc.subcore_barrier()

    pltpu.make_async_copy(eidx_hbm.at[wid, 0], i0, si0).wait()
    pltpu.async_copy(g_hbm.at[i0.at[0]], r0b, sg0)
    pltpu.make_async_copy(eidx_hbm.at[wid, 0], i1, si1).wait()
    pltpu.async_copy(g_hbm.at[i1.at[0]], r1b, sg1)

    def sstep(it, carry):
        j = 4 * it
        pltpu.make_async_copy(g_hbm.at[i0.at[0]], r0b, sg0).wait()
        pltpu.async_copy(r0b, acc.at[i0.at[1]], ss0, add=True)
        pltpu.make_async_copy(g_hbm.at[i1.at[0]], r1b, sg1).wait()
        pltpu.async_copy(r1b, acc.at[i1.at[1]], ss1, add=True)
        pltpu.make_async_copy(eidx_hbm.at[wid, 0], i2, si2).wait()
        pltpu.make_async_copy(r0b, acc.at[i0.at[1]], ss0).wait()
        pltpu.async_copy(g_hbm.at[i2.at[0]], r0b, sg0)

        @pl.when(j + 4 < NCHUNK)
        def _():
            pltpu.async_copy(eidx_hbm.at[wid, j + 4], i0, si0)

        pltpu.make_async_copy(eidx_hbm.at[wid, 0], i3, si3).wait()
        pltpu.make_async_copy(r1b, acc.at[i1.at[1]], ss1).wait()
        pltpu.async_copy(g_hbm.at[i3.at[0]], r1b, sg1)

        @pl.when(j + 5 < NCHUNK)
        def _():
            pltpu.async_copy(eidx_hbm.at[wid, j + 5], i1, si1)

        pltpu.make_async_copy(g_hbm.at[i2.at[0]], r0b, sg0).wait()
        pltpu.async_copy(r0b, acc.at[i2.at[1]], ss0, add=True)
        pltpu.make_async_copy(g_hbm.at[i3.at[0]], r1b, sg1).wait()
        pltpu.async_copy(r1b, acc.at[i3.at[1]], ss1, add=True)

        @pl.when(j + 4 < NCHUNK)
        def _():
            pltpu.make_async_copy(eidx_hbm.at[wid, 0], i0, si0).wait()

        pltpu.make_async_copy(r0b, acc.at[i2.at[1]], ss0).wait()

        @pl.when(j + 4 < NCHUNK)
        def _():
            pltpu.async_copy(g_hbm.at[i0.at[0]], r0b, sg0)

        @pl.when(j + 6 < NCHUNK)
        def _():
            pltpu.async_copy(eidx_hbm.at[wid, j + 6], i2, si2)

        @pl.when(j + 5 < NCHUNK)
        def _():
            pltpu.make_async_copy(eidx_hbm.at[wid, 0], i1, si1).wait()

        pltpu.make_async_copy(r1b, acc.at[i3.at[1]], ss1).wait()

        @pl.when(j + 5 < NCHUNK)
        def _():
            pltpu.async_copy(g_hbm.at[i1.at[0]], r1b, sg1)

        @pl.when(j + 7 < NCHUNK)
        def _():
            pltpu.async_copy(eidx_hbm.at[wid, j + 7], i3, si3)

        return carry

    lax.fori_loop(0, NCHUNK // 4, sstep, 0)
    plsc.subcore_barrier()
    pltpu.sync_copy(acc.at[pl.ds(rbase, ROWS_PER_TILE)],
                    out_hbm.at[c, pl.ds(rbase, ROWS_PER_TILE)])


def _tc_prep_body(x_ref, w_ref, deg_ref, dinv_ref, g_ref):
    deg = deg_ref[0, :] + deg_ref[1, :] + 1.0
    dinv = lax.rsqrt(deg)
    dinv_ref[...] = dinv
    xw = jnp.dot(x_ref[...], w_ref[...], preferred_element_type=jnp.float32)
    g_ref[...] = xw * dinv[:, None]


_tc_prep = pl.pallas_call(
    _tc_prep_body,
    grid=(NBLK,),
    in_specs=[
        pl.BlockSpec((BLK, D), lambda i: (i, 0)),
        pl.BlockSpec((D, D), lambda i: (0, 0)),
        pl.BlockSpec((NC, BLK), lambda i: (0, i)),
    ],
    out_specs=[
        pl.BlockSpec((BLK,), lambda i: (i,)),
        pl.BlockSpec((BLK, D), lambda i: (i, 0)),
    ],
    out_shape=[
        jax.ShapeDtypeStruct((N_PAD,), jnp.float32),
        jax.ShapeDtypeStruct((N_PAD, D), jnp.float32),
    ],
)


def _layer_head(parts_ref, gprev_ref, dinv_ref, b_ref, batch_ref):
    ssum = parts_ref[0] + parts_ref[1] + gprev_ref[...]
    dinv = dinv_ref[...][:, None]
    h = jnp.maximum(ssum * dinv + b_ref[...], 0.0)
    onehot = (batch_ref[...][:, None]
              == lax.broadcasted_iota(jnp.int32, (BLK, G), 1)).astype(jnp.float32)
    contrib = lax.dot_general(onehot, h, (((0,), (0,)), ((), ())),
                              preferred_element_type=jnp.float32)
    return h, dinv, contrib


def _tc_mid_body(parts_ref, gprev_ref, dinv_ref, b_ref, w_ref, batch_ref,
                 gnext_ref, pool_ref):
    i = pl.program_id(0)
    h, dinv, contrib = _layer_head(parts_ref, gprev_ref, dinv_ref, b_ref, batch_ref)
    gnext_ref[...] = jnp.dot(h, w_ref[...],
                             preferred_element_type=jnp.float32) * dinv

    @pl.when(i == 0)
    def _():
        pool_ref[...] = contrib

    @pl.when(i > 0)
    def _():
        pool_ref[...] += contrib


_tc_mid = pl.pallas_call(
    _tc_mid_body,
    grid=(NBLK,),
    in_specs=[
        pl.BlockSpec((NC, BLK, D), lambda i: (0, i, 0)),
        pl.BlockSpec((BLK, D), lambda i: (i, 0)),
        pl.BlockSpec((BLK,), lambda i: (i,)),
        pl.BlockSpec((1, D), lambda i: (0, 0)),
        pl.BlockSpec((D, D), lambda i: (0, 0)),
        pl.BlockSpec((BLK,), lambda i: (i,)),
    ],
    out_specs=[
        pl.BlockSpec((BLK, D), lambda i: (i, 0)),
        pl.BlockSpec((G, D), lambda i: (0, 0)),
    ],
    out_shape=[
        jax.ShapeDtypeStruct((N_PAD, D), jnp.float32),
        jax.ShapeDtypeStruct((G, D), jnp.float32),
    ],
)


def _tc_last_body(parts_ref, gprev_ref, dinv_ref, b_ref, batch_ref, pool_ref):
    i = pl.program_id(0)
    _, _, contrib = _layer_head(parts_ref, gprev_ref, dinv_ref, b_ref, batch_ref)

    @pl.when(i == 0)
    def _():
        pool_ref[...] = contrib

    @pl.when(i > 0)
    def _():
        pool_ref[...] += contrib


_tc_last = pl.pallas_call(
    _tc_last_body,
    grid=(NBLK,),
    in_specs=[
        pl.BlockSpec((NC, BLK, D), lambda i: (0, i, 0)),
        pl.BlockSpec((BLK, D), lambda i: (i, 0)),
        pl.BlockSpec((BLK,), lambda i: (i,)),
        pl.BlockSpec((1, D), lambda i: (0, 0)),
        pl.BlockSpec((BLK,), lambda i: (i,)),
    ],
    out_specs=pl.BlockSpec((G, D), lambda i: (0, 0)),
    out_shape=jax.ShapeDtypeStruct((G, D), jnp.float32),
)


def kernel(x, edge_index, batch, W1, b1, W2, b2, W3, b3):
    e32 = edge_index.astype(jnp.int32)
    pad_tgt = PAD_NODE + jnp.arange(E_PAD - E, dtype=jnp.int32) % (N_PAD - N_NODES)
    src = jnp.concatenate([e32[0], pad_tgt])
    dst = jnp.concatenate([e32[1], pad_tgt])
    eidx = jnp.stack([src.reshape(NW, NCHUNK, CHUNK),
                      dst.reshape(NW, NCHUNK, CHUNK)], axis=2)
    x_pad = jnp.pad(x, ((0, N_PAD - N_NODES), (0, 0)))
    batch_pad = jnp.pad(batch.astype(jnp.int32), (0, N_PAD - N_NODES),
                        constant_values=G)

    deg = _sc_degree(eidx)
    dinv, g1 = _tc_prep(x_pad, W1, deg)
    p1 = _sc_aggregate(g1, eidx)
    g2, pool1 = _tc_mid(p1, g1, dinv, b1.reshape(1, D), W2, batch_pad)
    p2 = _sc_aggregate(g2, eidx)
    g3, pool2 = _tc_mid(p2, g2, dinv, b2.reshape(1, D), W3, batch_pad)
    p3 = _sc_aggregate(g3, eidx)
    pool3 = _tc_last(p3, g3, dinv, b3.reshape(1, D), batch_pad)
    return jnp.concatenate([pool1, pool2, pool3], axis=1)


# speedup vs baseline: 1.3140x; 1.3140x over previous
import functools

import jax
import jax.numpy as jnp
from jax import lax
from jax.experimental import pallas as pl
from jax.experimental.pallas import tpu as pltpu
from jax.experimental.pallas import tpu_sc as plsc

N_NODES = 10000
N_PAD = 10240
D = 128
G = 128
E = 320000
NC = 2
NS = 16
NW = NC * NS
CHUNK = 128
NCHUNK = 80
E_PAD = NW * NCHUNK * CHUNK
PAD_NODE = N_NODES
ROWS_PER_TILE = N_PAD // NS
BLK = 512
NBLK = N_PAD // BLK

_mesh = plsc.VectorSubcoreMesh(core_axis_name="c", subcore_axis_name="s")


@functools.partial(
    pl.kernel,
    mesh=_mesh,
    out_type=jax.ShapeDtypeStruct((NC, N_PAD), jnp.float32),
    scratch_types=[
        pltpu.VMEM((NCHUNK, CHUNK), jnp.int32),
        pltpu.VMEM((128,), jnp.float32),
        pltpu.VMEM((ROWS_PER_TILE,), jnp.float32),
        pltpu.VMEM_SHARED((N_PAD,), jnp.float32),
        pltpu.SemaphoreType.DMA,
        pltpu.SemaphoreType.DMA,
    ],
)
def _sc_degree(dstq_hbm, out_hbm, idxs, ones_v, zero_v, acc, si, ss):
    c = lax.axis_index("c")
    s = lax.axis_index("s")
    wid = c * NS + s
    pltpu.async_copy(dstq_hbm.at[wid], idxs, si)
    for k in range(8):
        ones_v[pl.ds(k * 16, 16)] = jnp.ones((16,), jnp.float32)

    def zbody(i, carry):
        zero_v[pl.ds(i * 16, 16)] = jnp.zeros((16,), jnp.float32)
        return carry

    lax.fori_loop(0, ROWS_PER_TILE // 16, zbody, 0)
    r0 = s * ROWS_PER_TILE
    pltpu.sync_copy(zero_v, acc.at[pl.ds(r0, ROWS_PER_TILE)])
    plsc.subcore_barrier()
    pltpu.make_async_copy(dstq_hbm.at[wid], idxs, si).wait()

    def body(i, carry):
        j = 8 * i
        for k in range(8):
            pltpu.async_copy(ones_v.at[pl.ds(0, CHUNK)],
                             acc.at[idxs.at[j + k]], ss, add=True)
        for k in range(8):
            pltpu.make_async_copy(ones_v.at[pl.ds(0, CHUNK)],
                                  acc.at[idxs.at[j + k]], ss).wait()
        return carry

    lax.fori_loop(0, NCHUNK // 8, body, 0)
    plsc.subcore_barrier()
    pltpu.sync_copy(acc.at[pl.ds(r0, ROWS_PER_TILE)],
                    out_hbm.at[c, pl.ds(r0, ROWS_PER_TILE)])


NSLAB = NCHUNK // 8
NPAIR = NSLAB // 2


@functools.partial(
    pl.kernel,
    mesh=_mesh,
    out_type=jax.ShapeDtypeStruct((NC, N_PAD, D), jnp.float32),
    scratch_types=[
        pltpu.VMEM((8, 2, CHUNK), jnp.int32),
        pltpu.VMEM((8, 2, CHUNK), jnp.int32),
        pltpu.VMEM((CHUNK, D), jnp.float32),
        pltpu.VMEM((CHUNK, D), jnp.float32),
        pltpu.VMEM_SHARED((N_PAD, D), jnp.float32),
        pltpu.SemaphoreType.DMA,
        pltpu.SemaphoreType.DMA,
        pltpu.SemaphoreType.DMA,
        pltpu.SemaphoreType.DMA,
    ],
)
def _sc_aggregate(g_hbm, eidx_hbm, out_hbm, ia, ib, r0b, r1b, acc,
                  sg0, sg1, si_a, si_b):
    c = lax.axis_index("c")
    s = lax.axis_index("s")
    wid = c * NS + s
    pltpu.async_copy(eidx_hbm.at[wid, 0], ia, si_a)

    def zbody(i, carry):
        for k in range(8):
            r0b[i, pl.ds(k * 16, 16)] = jnp.zeros((16,), jnp.float32)
        return carry

    lax.fori_loop(0, CHUNK, zbody, 0)
    rbase = s * ROWS_PER_TILE
    for t in range(ROWS_PER_TILE // CHUNK):
        pltpu.sync_copy(r0b, acc.at[pl.ds(rbase + t * CHUNK, CHUNK)])
    plsc.subcore_barrier()

    pltpu.make_async_copy(eidx_hbm.at[wid, 0], ia, si_a).wait()
    pltpu.async_copy(eidx_hbm.at[wid, 1], ib, si_b)
    pltpu.async_copy(g_hbm.at[ia.at[0, 0]], r0b, sg0)

    rows = (r0b, r1b)
    gsems = (sg0, sg1)

    def pair(sp, carry):
        for k in range(16):
            slab, kk = (ia, k) if k < 8 else (ib, k - 8)
            cur, csem = rows[k % 2], gsems[k % 2]
            nxt, nsem = rows[(k + 1) % 2], gsems[(k + 1) % 2]
            if k == 7:
                pltpu.make_async_copy(eidx_hbm.at[wid, 0], ib, si_b).wait()
            if k < 15:
                nslab, nkk = (ia, k + 1) if k + 1 < 8 else (ib, k - 7)
                pltpu.async_copy(g_hbm.at[nslab.at[nkk, 0]], nxt, nsem)
            else:
                @pl.when(sp < NPAIR - 1)
                def _():
                    pltpu.make_async_copy(eidx_hbm.at[wid, 0], ia, si_a).wait()
                    pltpu.async_copy(g_hbm.at[ia.at[0, 0]], nxt, nsem)

            pltpu.make_async_copy(g_hbm.at[slab.at[kk, 0]], cur, csem).wait()
            pltpu.sync_copy(cur, acc.at[slab.at[kk, 1]], add=True)

            if k == 7:
                @pl.when(sp < NPAIR - 1)
                def _():
                    pltpu.async_copy(eidx_hbm.at[wid, 2 * sp + 2], ia, si_a)
            if k == 15:
                @pl.when(sp < NPAIR - 1)
                def _():
                    pltpu.async_copy(eidx_hbm.at[wid, 2 * sp + 3], ib, si_b)
        return carry

    lax.fori_loop(0, NPAIR, pair, 0)
    plsc.subcore_barrier()
    pltpu.sync_copy(acc.at[pl.ds(rbase, ROWS_PER_TILE)],
                    out_hbm.at[c, pl.ds(rbase, ROWS_PER_TILE)])


def _tc_prep_body(x_ref, w_ref, deg_ref, dinv_ref, g_ref):
    deg = deg_ref[0, :] + deg_ref[1, :] + 1.0
    dinv = lax.rsqrt(deg)
    dinv_ref[...] = dinv
    xw = jnp.dot(x_ref[...], w_ref[...], preferred_element_type=jnp.float32)
    g_ref[...] = xw * dinv[:, None]


_tc_prep = pl.pallas_call(
    _tc_prep_body,
    grid=(NBLK,),
    in_specs=[
        pl.BlockSpec((BLK, D), lambda i: (i, 0)),
        pl.BlockSpec((D, D), lambda i: (0, 0)),
        pl.BlockSpec((NC, BLK), lambda i: (0, i)),
    ],
    out_specs=[
        pl.BlockSpec((BLK,), lambda i: (i,)),
        pl.BlockSpec((BLK, D), lambda i: (i, 0)),
    ],
    out_shape=[
        jax.ShapeDtypeStruct((N_PAD,), jnp.float32),
        jax.ShapeDtypeStruct((N_PAD, D), jnp.float32),
    ],
)


def _layer_head(parts_ref, gprev_ref, dinv_ref, b_ref, batch_ref):
    ssum = parts_ref[0] + parts_ref[1] + gprev_ref[...]
    dinv = dinv_ref[...][:, None]
    h = jnp.maximum(ssum * dinv + b_ref[...], 0.0)
    onehot = (batch_ref[...][:, None]
              == lax.broadcasted_iota(jnp.int32, (BLK, G), 1)).astype(jnp.float32)
    contrib = lax.dot_general(onehot, h, (((0,), (0,)), ((), ())),
                              preferred_element_type=jnp.float32)
    return h, dinv, contrib


def _tc_mid_body(parts_ref, gprev_ref, dinv_ref, b_ref, w_ref, batch_ref,
                 gnext_ref, pool_ref):
    i = pl.program_id(0)
    h, dinv, contrib = _layer_head(parts_ref, gprev_ref, dinv_ref, b_ref, batch_ref)
    gnext_ref[...] = jnp.dot(h, w_ref[...],
                             preferred_element_type=jnp.float32) * dinv

    @pl.when(i == 0)
    def _():
        pool_ref[...] = contrib

    @pl.when(i > 0)
    def _():
        pool_ref[...] += contrib


_tc_mid = pl.pallas_call(
    _tc_mid_body,
    grid=(NBLK,),
    in_specs=[
        pl.BlockSpec((NC, BLK, D), lambda i: (0, i, 0)),
        pl.BlockSpec((BLK, D), lambda i: (i, 0)),
        pl.BlockSpec((BLK,), lambda i: (i,)),
        pl.BlockSpec((1, D), lambda i: (0, 0)),
        pl.BlockSpec((D, D), lambda i: (0, 0)),
        pl.BlockSpec((BLK,), lambda i: (i,)),
    ],
    out_specs=[
        pl.BlockSpec((BLK, D), lambda i: (i, 0)),
        pl.BlockSpec((G, D), lambda i: (0, 0)),
    ],
    out_shape=[
        jax.ShapeDtypeStruct((N_PAD, D), jnp.float32),
        jax.ShapeDtypeStruct((G, D), jnp.float32),
    ],
)


def _tc_last_body(parts_ref, gprev_ref, dinv_ref, b_ref, batch_ref, pool_ref):
    i = pl.program_id(0)
    _, _, contrib = _layer_head(parts_ref, gprev_ref, dinv_ref, b_ref, batch_ref)

    @pl.when(i == 0)
    def _():
        pool_ref[...] = contrib

    @pl.when(i > 0)
    def _():
        pool_ref[...] += contrib


_tc_last = pl.pallas_call(
    _tc_last_body,
    grid=(NBLK,),
    in_specs=[
        pl.BlockSpec((NC, BLK, D), lambda i: (0, i, 0)),
        pl.BlockSpec((BLK, D), lambda i: (i, 0)),
        pl.BlockSpec((BLK,), lambda i: (i,)),
        pl.BlockSpec((1, D), lambda i: (0, 0)),
        pl.BlockSpec((BLK,), lambda i: (i,)),
    ],
    out_specs=pl.BlockSpec((G, D), lambda i: (0, 0)),
    out_shape=jax.ShapeDtypeStruct((G, D), jnp.float32),
)


def kernel(x, edge_index, batch, W1, b1, W2, b2, W3, b3):
    e32 = edge_index.astype(jnp.int32)
    pad_tgt = PAD_NODE + jnp.arange(E_PAD - E, dtype=jnp.int32) % (N_PAD - N_NODES)
    src = jnp.concatenate([e32[0], pad_tgt])
    dst = jnp.concatenate([e32[1], pad_tgt])
    dstq = dst.reshape(NW, NCHUNK, CHUNK)
    eidx = jnp.stack([src.reshape(NW, NCHUNK, CHUNK), dstq],
                     axis=2).reshape(NW, NSLAB, 8, 2, CHUNK)
    x_pad = jnp.pad(x, ((0, N_PAD - N_NODES), (0, 0)))
    batch_pad = jnp.pad(batch.astype(jnp.int32), (0, N_PAD - N_NODES),
                        constant_values=G)

    deg = _sc_degree(dstq)
    dinv, g1 = _tc_prep(x_pad, W1, deg)
    p1 = _sc_aggregate(g1, eidx)
    g2, pool1 = _tc_mid(p1, g1, dinv, b1.reshape(1, D), W2, batch_pad)
    p2 = _sc_aggregate(g2, eidx)
    g3, pool2 = _tc_mid(p2, g2, dinv, b2.reshape(1, D), W3, batch_pad)
    p3 = _sc_aggregate(g3, eidx)
    pool3 = _tc_last(p3, g3, dinv, b3.reshape(1, D), batch_pad)
    return jnp.concatenate([pool1, pool2, pool3], axis=1)
